# baseline (speedup 1.0000x reference)
import functools

import jax
import jax.numpy as jnp
from jax.experimental import pallas as pl
from jax.experimental.pallas import tpu as pltpu

_KEEP_RATIO = 0.25
_MIN_TOPK = 4


def _fused_kernel(x_ref, w1_ref, beta_ref, w2r_ref, b2_ref,
                  score_ref, v_ref, taps_ref, *, H, W, K, CS):
    n = H * W
    nch = n // CS
    w1 = w1_ref[...]
    beta = beta_ref[...]
    w2r = w2r_ref[...]

    for ci in range(nch):
        sl = slice(ci * CS, (ci + 1) * CS)
        xc = x_ref[0, :, sl]
        hid = jax.lax.dot_general(w1, xc, (((1,), (0,)), ((), ())),
                                  preferred_element_type=jnp.float32)
        hid = jnp.maximum(hid + beta, 0.0)
        taps_ref[:, sl] = jax.lax.dot_general(
            w2r, hid, (((1,), (0,)), ((), ())),
            preferred_element_type=jnp.float32)
    taps = taps_ref[...]

    col = jax.lax.broadcasted_iota(jnp.int32, (1, n), 1) % W
    m_left = (col >= 1).astype(jnp.float32)
    m_right = (col <= W - 2).astype(jnp.float32)

    def shiftrow(v, s):
        if s == 0:
            return v
        z = jnp.zeros((1, abs(s)), jnp.float32)
        if s > 0:
            return jnp.concatenate([v[:, s:], z], axis=1)
        return jnp.concatenate([z, v[:, :n + s]], axis=1)

    score_row = jnp.zeros((1, n), jnp.float32)
    for kh in range(3):
        for kw in range(3):
            dy, dx = kh - 1, kw - 1
            sh = shiftrow(taps[kh * 3 + kw:kh * 3 + kw + 1, :], dy * W + dx)
            if dx == 1:
                sh = sh * m_right
            elif dx == -1:
                sh = sh * m_left
            score_row = score_row + sh
    score_row = score_row + b2_ref[0, 0]
    score_ref[...] = score_row.reshape(1, 1, n)

    sbits = jax.lax.bitcast_convert_type(score_row, jnp.uint32)
    top = jnp.uint32(0x80000000)
    key = jnp.where((sbits & top) != 0, ~sbits, sbits | top)

    js = jax.lax.broadcasted_iota(jnp.uint32, (15, 1), 0) + jnp.uint32(1)
    thr = jnp.uint32(0)
    for r in range(8):
        shift = jnp.uint32(28 - 4 * r)
        cands = thr | jax.lax.shift_left(js, shift)
        cnts = jnp.sum((key >= cands).astype(jnp.int32), axis=1,
                       keepdims=True)
        j = jnp.sum((cnts >= K).astype(jnp.int32)).astype(jnp.uint32)
        thr = thr | jax.lax.shift_left(j, shift)

    gt = key > thr
    eq = key == thr
    cgt = jnp.sum(gt.astype(jnp.int32))
    ceq = jnp.sum(eq.astype(jnp.int32))
    frac = (K - cgt).astype(jnp.float32) / ceq.astype(jnp.float32)
    m = jnp.max(score_row)
    e = jnp.exp(score_row - m)
    wsel = e * (gt.astype(jnp.float32) + frac * eq.astype(jnp.float32))
    denom = jnp.sum(wsel)
    acc = jnp.zeros((1, w1.shape[1]), jnp.float32)
    for ci in range(nch):
        sl = slice(ci * CS, (ci + 1) * CS)
        acc = acc + jax.lax.dot_general(
            wsel[:, sl], x_ref[0, :, sl], (((1,), (1,)), ((), ())),
            precision=jax.lax.Precision.HIGHEST,
            preferred_element_type=jnp.float32)
    v_ref[...] = (acc / denom).reshape(1, 1, -1)


def kernel(x, conv1_w, bn_gamma, bn_beta, conv2_w, conv2_b):
    B, C, H, W = x.shape
    n = H * W
    HID = conv1_w.shape[0]
    K = min(max(_MIN_TOPK, int(n * _KEEP_RATIO)), n)

    x2 = x.reshape(B, C, n)
    scale = (bn_gamma / jnp.sqrt(jnp.float32(1.0 + 1e-5))).reshape(HID, 1)
    w1 = conv1_w.reshape(HID, C) * scale
    beta = bn_beta.reshape(HID, 1)
    w2r = conv2_w.reshape(HID, 9).T
    b2 = conv2_b.reshape(1, 1)

    CS = n // 8 if n % 8 == 0 else n
    score, v = pl.pallas_call(
        functools.partial(_fused_kernel, H=H, W=W, K=K, CS=CS),
        grid=(B,),
        in_specs=[
            pl.BlockSpec((1, C, n), lambda b: (b, 0, 0)),
            pl.BlockSpec((HID, C), lambda b: (0, 0)),
            pl.BlockSpec((HID, 1), lambda b: (0, 0)),
            pl.BlockSpec((9, HID), lambda b: (0, 0)),
            pl.BlockSpec((1, 1), lambda b: (0, 0)),
        ],
        out_specs=[
            pl.BlockSpec((1, 1, n), lambda b: (b, 0, 0)),
            pl.BlockSpec((1, 1, C), lambda b: (b, 0, 0)),
        ],
        out_shape=[
            jax.ShapeDtypeStruct((B, 1, n), jnp.float32),
            jax.ShapeDtypeStruct((B, 1, C), jnp.float32),
        ],
        scratch_shapes=[pltpu.VMEM((9, n), jnp.float32)],
    )(x2, w1, beta, w2r, b2)

    return (v.reshape(B, C), score.reshape(B, 1, H, W))


# speedup vs baseline: 1.3942x; 1.3942x over previous
import functools

import jax
import jax.numpy as jnp
from jax.experimental import pallas as pl
from jax.experimental.pallas import tpu as pltpu

_KEEP_RATIO = 0.25
_MIN_TOPK = 4


def _fused_kernel(x_ref, w1_ref, scale_ref, beta_ref, w2r_ref, b2_ref,
                  score_ref, v_ref, taps_ref, *, H, W, K, CS):
    n = H * W
    nch = n // CS
    w1 = w1_ref[...]
    scale = scale_ref[...]
    beta = beta_ref[...]
    w2r = w2r_ref[...]

    for ci in range(nch):
        sl = slice(ci * CS, (ci + 1) * CS)
        xc = x_ref[0, :, sl]
        hid = jax.lax.dot_general(w1, xc, (((1,), (0,)), ((), ())),
                                  preferred_element_type=jnp.float32)
        hid = jnp.maximum(hid * scale + beta, 0.0)
        taps_ref[:, sl] = jax.lax.dot_general(
            w2r, hid, (((1,), (0,)), ((), ())),
            preferred_element_type=jnp.float32)
    taps = taps_ref[...]

    col = jax.lax.broadcasted_iota(jnp.int32, (1, n), 1) % W
    m_left = (col >= 1).astype(jnp.float32)
    m_right = (col <= W - 2).astype(jnp.float32)

    def shiftrow(v, s):
        if s == 0:
            return v
        z = jnp.zeros((1, abs(s)), jnp.float32)
        if s > 0:
            return jnp.concatenate([v[:, s:], z], axis=1)
        return jnp.concatenate([z, v[:, :n + s]], axis=1)

    score_row = jnp.zeros((1, n), jnp.float32)
    for kh in range(3):
        for kw in range(3):
            dy, dx = kh - 1, kw - 1
            sh = shiftrow(taps[kh * 3 + kw:kh * 3 + kw + 1, :], dy * W + dx)
            if dx == 1:
                sh = sh * m_right
            elif dx == -1:
                sh = sh * m_left
            score_row = score_row + sh
    score_row = score_row + b2_ref[0, 0]
    score_ref[...] = score_row.reshape(1, 1, n)

    sbits = jax.lax.bitcast_convert_type(score_row, jnp.uint32)
    top = jnp.uint32(0x80000000)
    key = jnp.where((sbits & top) != 0, ~sbits, sbits | top)

    js = jax.lax.broadcasted_iota(jnp.uint32, (15, 1), 0) + jnp.uint32(1)
    thr = jnp.uint32(0)
    for r in range(8):
        shift = jnp.uint32(28 - 4 * r)
        cands = thr | jax.lax.shift_left(js, shift)
        cnts = jnp.sum((key >= cands).astype(jnp.int32), axis=1,
                       keepdims=True)
        j = jnp.sum((cnts >= K).astype(jnp.int32)).astype(jnp.uint32)
        thr = thr | jax.lax.shift_left(j, shift)

    gt = key > thr
    eq = key == thr
    cgt = jnp.sum(gt.astype(jnp.int32))
    ceq = jnp.sum(eq.astype(jnp.int32))
    frac = (K - cgt).astype(jnp.float32) / ceq.astype(jnp.float32)
    m = jnp.max(score_row)
    e = jnp.exp(score_row - m)
    wsel = e * (gt.astype(jnp.float32) + frac * eq.astype(jnp.float32))
    denom = jnp.sum(wsel)
    acc = jnp.zeros((w1.shape[1],), jnp.float32)
    for ci in range(nch):
        sl = slice(ci * CS, (ci + 1) * CS)
        acc = acc + jnp.sum(x_ref[0, :, sl] * wsel[:, sl], axis=1)
    v_ref[...] = (acc / denom).reshape(1, 1, -1)


def kernel(x, conv1_w, bn_gamma, bn_beta, conv2_w, conv2_b):
    B, C, H, W = x.shape
    n = H * W
    HID = conv1_w.shape[0]
    K = min(max(_MIN_TOPK, int(n * _KEEP_RATIO)), n)

    x2 = x.reshape(B, C, n)
    w1 = conv1_w.reshape(HID, C)
    scale = (bn_gamma / jnp.sqrt(jnp.float32(1.0 + 1e-5))).reshape(HID, 1)
    beta = bn_beta.reshape(HID, 1)
    w2r = conv2_w.reshape(HID, 9).T
    b2 = conv2_b.reshape(1, 1)

    CS = n // 8 if n % 8 == 0 else n
    score, v = pl.pallas_call(
        functools.partial(_fused_kernel, H=H, W=W, K=K, CS=CS),
        grid=(B,),
        in_specs=[
            pl.BlockSpec((1, C, n), lambda b: (b, 0, 0)),
            pl.BlockSpec((HID, C), lambda b: (0, 0)),
            pl.BlockSpec((HID, 1), lambda b: (0, 0)),
            pl.BlockSpec((HID, 1), lambda b: (0, 0)),
            pl.BlockSpec((9, HID), lambda b: (0, 0)),
            pl.BlockSpec((1, 1), lambda b: (0, 0)),
        ],
        out_specs=[
            pl.BlockSpec((1, 1, n), lambda b: (b, 0, 0)),
            pl.BlockSpec((1, 1, C), lambda b: (b, 0, 0)),
        ],
        out_shape=[
            jax.ShapeDtypeStruct((B, 1, n), jnp.float32),
            jax.ShapeDtypeStruct((B, 1, C), jnp.float32),
        ],
        scratch_shapes=[pltpu.VMEM((9, n), jnp.float32)],
    )(x2, w1, scale, beta, w2r, b2)

    return (v.reshape(B, C), score.reshape(B, 1, H, W))
